# baseline (speedup 1.0000x reference)
import functools

import jax
import jax.numpy as jnp
from jax import lax
from jax.experimental import pallas as pl
from jax.experimental.pallas import tpu as pltpu
from jax.experimental.pallas import tpu_sc as plsc

F_FIELDS = 26
VOCAB = 100000
D = 64
B = 1024
IN_DIM = F_FIELDS * D
N_ROWS = B * F_FIELDS

_NC = 2
_NS = 16
_NW = _NC * _NS
_RPW = N_ROWS // _NW
_CHUNK = 104
_NCHUNK = _RPW // _CHUNK

_WIN = 13
_OBUF = 4


def _sc_gather_body(rpw, idx_hbm, t3_hbm, out_hbm, idx_v, outbuf, blks,
                    sem_out, *sems):
    ngrp = rpw // _WIN
    wid = lax.axis_index("s") * _NC + lax.axis_index("c")
    base = wid * rpw

    pltpu.sync_copy(idx_hbm.at[pl.ds(base, rpw)], idx_v.at[pl.ds(0, rpw)])

    d_idx = [lax.iota(jnp.int32, 16) + (16 * k) for k in range(D // 16)]

    def _fetch(j, w):
        v = idx_v[pl.ds(j, 16)][0]
        f = lax.rem(base + j, F_FIELDS)
        voff = pl.multiple_of((v >> 7) << 7, 128)
        pltpu.make_async_copy(
            t3_hbm.at[f, :, pl.ds(voff, 128)], blks.at[w], sems[w]).start()

    for w in range(_WIN):
        _fetch(w, w)

    def _group(g, carry):
        jb = g * _WIN
        obase = lax.rem(g, _OBUF) * (_WIN * D)
        for w in range(_WIN):
            j = jb + w
            pltpu.make_async_copy(
                t3_hbm.at[0, :, pl.ds(0, 128)], blks.at[w], sems[w]).wait()
            v = idx_v[pl.ds(j, 16)][0]
            lane = jnp.full((16,), v & 127, dtype=jnp.int32)
            for k in range(D // 16):
                col = plsc.load_gather(blks.at[w], [d_idx[k], lane])
                outbuf[pl.ds(obase + w * D + 16 * k, 16)] = col

            @pl.when(g < ngrp - 1)
            def _():
                _fetch(j + _WIN, w)

        @pl.when(g >= _OBUF - 1)
        def _():
            pltpu.make_async_copy(
                out_hbm.at[pl.ds(0, _WIN * D)],
                outbuf.at[pl.ds(0, _WIN * D)], sem_out).wait()

        pltpu.make_async_copy(
            outbuf.at[pl.ds(obase, _WIN * D)],
            out_hbm.at[pl.ds((base + jb) * D, _WIN * D)], sem_out).start()
        return carry

    lax.fori_loop(0, ngrp, _group, 0)

    for _ in range(_OBUF - 1):
        pltpu.make_async_copy(
            out_hbm.at[pl.ds(0, _WIN * D)],
            outbuf.at[pl.ds(0, _WIN * D)], sem_out).wait()


@functools.lru_cache(maxsize=2)
def _make_sc_gather(n_rows):
    rpw = n_rows // _NW
    mesh = plsc.VectorSubcoreMesh(
        core_axis_name="c", subcore_axis_name="s", num_cores=_NC)
    return pl.kernel(
        functools.partial(_sc_gather_body, rpw),
        mesh=mesh,
        compiler_params=pltpu.CompilerParams(needs_layout_passes=False),
        out_type=jax.ShapeDtypeStruct((n_rows * D,), jnp.float32),
        scratch_types=[
            pltpu.VMEM((rpw + 16,), jnp.int32),
            pltpu.VMEM((_OBUF * _WIN * D,), jnp.float32),
            pltpu.VMEM((_WIN, D, 128), jnp.float32),
            pltpu.SemaphoreType.DMA,
        ] + [pltpu.SemaphoreType.DMA] * _WIN,
    )


def _tc_body(c_ref, wk_ref, w1_ref, b1_ref, w2_ref, b2_ref, w3_ref, b3_ref,
             w4_ref, b4_ref, o_ref):
    c = c_ref[...]
    z = jnp.dot(c, wk_ref[...], preferred_element_type=jnp.float32)
    fw = 0.5 * jnp.sum(c * z, axis=1, keepdims=True)
    h = jnp.maximum(
        jnp.dot(c, w1_ref[...], preferred_element_type=jnp.float32)
        + b1_ref[...], 0.0)
    h = jnp.maximum(
        jnp.dot(h, w2_ref[...], preferred_element_type=jnp.float32)
        + b2_ref[...], 0.0)
    h = jnp.maximum(
        jnp.dot(h, w3_ref[...], preferred_element_type=jnp.float32)
        + b3_ref[...], 0.0)
    m = jnp.dot(h, w4_ref[...], preferred_element_type=jnp.float32) + b4_ref[...]
    o_ref[...] = jax.nn.sigmoid(fw + m)


_BB = 256


def _tc_call(c, wk, W1, b1, W2, b2, W3, b3, W4, b4):
    nb = c.shape[0]
    full = lambda i: (0, 0)
    return pl.pallas_call(
        _tc_body,
        grid=(nb // _BB,),
        in_specs=[
            pl.BlockSpec((_BB, IN_DIM), lambda i: (i, 0)),
            pl.BlockSpec((IN_DIM, IN_DIM), full),
            pl.BlockSpec((IN_DIM, 512), full),
            pl.BlockSpec((1, 512), full),
            pl.BlockSpec((512, 256), full),
            pl.BlockSpec((1, 256), full),
            pl.BlockSpec((256, 128), full),
            pl.BlockSpec((1, 128), full),
            pl.BlockSpec((128, 1), full),
            pl.BlockSpec((1, 1), full),
        ],
        out_specs=pl.BlockSpec((_BB, 1), lambda i: (i, 0)),
        out_shape=jax.ShapeDtypeStruct((nb, 1), jnp.float32),
    )(c, wk, W1, b1, W2, b2, W3, b3, W4, b4)


def kernel(x, tables, field_cov_W, W1, b1, W2, b2, W3, b3, W4, b4):
    idx_raw = x.astype(jnp.int32).reshape(-1)
    t3 = jnp.transpose(tables, (0, 2, 1))

    w_sym = (field_cov_W + field_cov_W.T) * 0.5
    w0 = w_sym * (1.0 - jnp.eye(F_FIELDS, dtype=jnp.float32))
    wk = (w0[:, None, :, None]
          * jnp.eye(D, dtype=jnp.float32)[None, :, None, :]
          ).reshape(IN_DIM, IN_DIM)

    half = N_ROWS // 2
    gather = _make_sc_gather(half)
    outs = []
    for h in range(2):
        g = gather(idx_raw[h * half:(h + 1) * half], t3)
        c = g.reshape(B // 2, IN_DIM)
        outs.append(
            _tc_call(c, wk, W1, b1.reshape(1, -1), W2, b2.reshape(1, -1),
                     W3, b3.reshape(1, -1), W4, b4.reshape(1, -1)))
    return jnp.concatenate(outs, axis=0).reshape(B)


# speedup vs baseline: 1.0372x; 1.0372x over previous
import functools

import jax
import jax.numpy as jnp
from jax import lax
from jax.experimental import pallas as pl
from jax.experimental.pallas import tpu as pltpu
from jax.experimental.pallas import tpu_sc as plsc

F_FIELDS = 26
VOCAB = 100000
D = 64
B = 1024
IN_DIM = F_FIELDS * D
N_ROWS = B * F_FIELDS

_NC = 2
_NS = 16
_NW = _NC * _NS
_RPW = N_ROWS // _NW
_CHUNK = 104
_NCHUNK = _RPW // _CHUNK

_WIN = 13
_SBUF = 2


def _sc_gather_body(rpw, idx_hbm, t3_hbm, out_hbm, idx_v, outbuf, blks,
                    sem_out, *sems):
    ngrp = rpw // _WIN
    wid = lax.axis_index("s") * _NC + lax.axis_index("c")
    base = wid * rpw

    pltpu.sync_copy(idx_hbm.at[pl.ds(base, rpw)], idx_v.at[pl.ds(0, rpw)])

    d_idx = [lax.iota(jnp.int32, 16) + (16 * k) for k in range(D // 16)]

    def _fetch(j, w):
        v = idx_v[pl.ds(j, 16)][0]
        f = lax.rem(base + j, F_FIELDS)
        voff = pl.multiple_of((v >> 7) << 7, 128)
        pltpu.make_async_copy(
            t3_hbm.at[f, :, pl.ds(voff, 128)], blks.at[w], sems[w]).start()

    for w in range(_WIN):
        _fetch(w, w)

    def _group(g, carry):
        jb = g * _WIN
        obase = (lax.rem(g // 2, _SBUF) * IN_DIM
                 + lax.rem(g, 2) * (_WIN * D))

        @pl.when(jnp.logical_and(lax.rem(g, 2) == 0, g >= 2 * _SBUF))
        def _():
            pltpu.make_async_copy(
                out_hbm.at[0, :], outbuf.at[pl.ds(0, IN_DIM)],
                sem_out).wait()
        for w in range(_WIN):
            j = jb + w
            pltpu.make_async_copy(
                t3_hbm.at[0, :, pl.ds(0, 128)], blks.at[w], sems[w]).wait()
            v = idx_v[pl.ds(j, 16)][0]
            lane = jnp.full((16,), v & 127, dtype=jnp.int32)
            for k in range(D // 16):
                col = plsc.load_gather(blks.at[w], [d_idx[k], lane])
                outbuf[pl.ds(obase + w * D + 16 * k, 16)] = col

            @pl.when(g < ngrp - 1)
            def _():
                _fetch(j + _WIN, w)

        @pl.when(lax.rem(g, 2) == 1)
        def _():
            s = base // F_FIELDS + (g - 1) // 2
            sbase = lax.rem((g - 1) // 2, _SBUF) * IN_DIM
            pltpu.make_async_copy(
                outbuf.at[pl.ds(sbase, IN_DIM)], out_hbm.at[s, :],
                sem_out).start()

        return carry

    lax.fori_loop(0, ngrp, _group, 0)

    for _ in range(_SBUF):
        pltpu.make_async_copy(
            out_hbm.at[0, :], outbuf.at[pl.ds(0, IN_DIM)], sem_out).wait()


@functools.lru_cache(maxsize=2)
def _make_sc_gather(n_rows):
    rpw = n_rows // _NW
    mesh = plsc.VectorSubcoreMesh(
        core_axis_name="c", subcore_axis_name="s", num_cores=_NC)
    return pl.kernel(
        functools.partial(_sc_gather_body, rpw),
        mesh=mesh,
        compiler_params=pltpu.CompilerParams(needs_layout_passes=False),
        out_type=jax.ShapeDtypeStruct((n_rows // F_FIELDS, IN_DIM),
                                      jnp.float32),
        scratch_types=[
            pltpu.VMEM((rpw + 16,), jnp.int32),
            pltpu.VMEM((_SBUF * IN_DIM,), jnp.float32),
            pltpu.VMEM((_WIN, D, 128), jnp.float32),
            pltpu.SemaphoreType.DMA,
        ] + [pltpu.SemaphoreType.DMA] * _WIN,
    )


def _tc_body(c_ref, wk_ref, w1_ref, b1_ref, w2_ref, b2_ref, w3_ref, b3_ref,
             w4_ref, b4_ref, o_ref):
    c = c_ref[...]
    z = jnp.dot(c, wk_ref[...], preferred_element_type=jnp.float32)
    fw = 0.5 * jnp.sum(c * z, axis=1, keepdims=True)
    h = jnp.maximum(
        jnp.dot(c, w1_ref[...], preferred_element_type=jnp.float32)
        + b1_ref[...], 0.0)
    h = jnp.maximum(
        jnp.dot(h, w2_ref[...], preferred_element_type=jnp.float32)
        + b2_ref[...], 0.0)
    h = jnp.maximum(
        jnp.dot(h, w3_ref[...], preferred_element_type=jnp.float32)
        + b3_ref[...], 0.0)
    m = jnp.dot(h, w4_ref[...], preferred_element_type=jnp.float32) + b4_ref[...]
    o_ref[...] = jax.nn.sigmoid(fw + m)


_BB = 256


def _tc_call(c, wk, W1, b1, W2, b2, W3, b3, W4, b4):
    nb = c.shape[0]
    full = lambda i: (0, 0)
    return pl.pallas_call(
        _tc_body,
        grid=(nb // _BB,),
        in_specs=[
            pl.BlockSpec((_BB, IN_DIM), lambda i: (i, 0)),
            pl.BlockSpec((IN_DIM, IN_DIM), full),
            pl.BlockSpec((IN_DIM, 512), full),
            pl.BlockSpec((1, 512), full),
            pl.BlockSpec((512, 256), full),
            pl.BlockSpec((1, 256), full),
            pl.BlockSpec((256, 128), full),
            pl.BlockSpec((1, 128), full),
            pl.BlockSpec((128, 1), full),
            pl.BlockSpec((1, 1), full),
        ],
        out_specs=pl.BlockSpec((_BB, 1), lambda i: (i, 0)),
        out_shape=jax.ShapeDtypeStruct((nb, 1), jnp.float32),
    )(c, wk, W1, b1, W2, b2, W3, b3, W4, b4)


def kernel(x, tables, field_cov_W, W1, b1, W2, b2, W3, b3, W4, b4):
    idx_raw = x.astype(jnp.int32).reshape(-1)
    t3 = jnp.transpose(tables, (0, 2, 1))

    w_sym = (field_cov_W + field_cov_W.T) * 0.5
    w0 = w_sym * (1.0 - jnp.eye(F_FIELDS, dtype=jnp.float32))
    wk = (w0[:, None, :, None]
          * jnp.eye(D, dtype=jnp.float32)[None, :, None, :]
          ).reshape(IN_DIM, IN_DIM)

    c = _make_sc_gather(N_ROWS)(idx_raw, t3)
    out = _tc_call(c, wk, W1, b1.reshape(1, -1), W2, b2.reshape(1, -1),
                   W3, b3.reshape(1, -1), W4, b4.reshape(1, -1))
    return out.reshape(B)


# speedup vs baseline: 1.0391x; 1.0018x over previous
import functools

import jax
import jax.numpy as jnp
from jax import lax
from jax.experimental import pallas as pl
from jax.experimental.pallas import tpu as pltpu
from jax.experimental.pallas import tpu_sc as plsc

F_FIELDS = 26
VOCAB = 100000
D = 64
B = 1024
IN_DIM = F_FIELDS * D
N_ROWS = B * F_FIELDS

_NC = 2
_NS = 16
_NW = _NC * _NS

_WIN = 13
_SBUF = 2


def _sc_gather_body(rpw, idx_hbm, t3_hbm, out_hbm, idx_v, outbuf, blks,
                    sem_out, *sems):
    ngrp = rpw // _WIN
    wid = lax.axis_index("s") * _NC + lax.axis_index("c")
    base = wid * rpw

    pltpu.sync_copy(idx_hbm.at[pl.ds(base, rpw)], idx_v.at[pl.ds(0, rpw)])

    d_idx = [lax.iota(jnp.int32, 16) + (16 * k) for k in range(D // 16)]

    def _fetch(j, w):
        v = idx_v[pl.ds(j, 16)][0]
        f = lax.rem(base + j, F_FIELDS)
        voff = pl.multiple_of((v >> 7) << 7, 128)
        pltpu.make_async_copy(
            t3_hbm.at[f, :, pl.ds(voff, 128)], blks.at[w], sems[w]).start()

    for w in range(_WIN):
        _fetch(w, w)

    def _group(g, carry):
        jb = g * _WIN
        obase = (lax.rem(g // 2, _SBUF) * IN_DIM
                 + lax.rem(g, 2) * (_WIN * D))

        @pl.when(jnp.logical_and(lax.rem(g, 2) == 0, g >= 2 * _SBUF))
        def _():
            pltpu.make_async_copy(
                out_hbm.at[0, :], outbuf.at[pl.ds(0, IN_DIM)],
                sem_out).wait()
        for w in range(_WIN):
            j = jb + w
            pltpu.make_async_copy(
                t3_hbm.at[0, :, pl.ds(0, 128)], blks.at[w], sems[w]).wait()
            v = idx_v[pl.ds(j, 16)][0]
            lane = jnp.full((16,), v & 127, dtype=jnp.int32)
            for k in range(D // 16):
                col = plsc.load_gather(blks.at[w], [d_idx[k], lane])
                outbuf[pl.ds(obase + w * D + 16 * k, 16)] = col

            @pl.when(g < ngrp - 1)
            def _():
                _fetch(j + _WIN, w)

        @pl.when(lax.rem(g, 2) == 1)
        def _():
            s = base // F_FIELDS + (g - 1) // 2
            sbase = lax.rem((g - 1) // 2, _SBUF) * IN_DIM
            pltpu.make_async_copy(
                outbuf.at[pl.ds(sbase, IN_DIM)], out_hbm.at[s, :],
                sem_out).start()

        return carry

    lax.fori_loop(0, ngrp, _group, 0)

    for _ in range(_SBUF):
        pltpu.make_async_copy(
            out_hbm.at[0, :], outbuf.at[pl.ds(0, IN_DIM)], sem_out).wait()


@functools.lru_cache(maxsize=2)
def _make_sc_gather(n_rows):
    rpw = n_rows // _NW
    mesh = plsc.VectorSubcoreMesh(
        core_axis_name="c", subcore_axis_name="s", num_cores=_NC)
    return pl.kernel(
        functools.partial(_sc_gather_body, rpw),
        mesh=mesh,
        compiler_params=pltpu.CompilerParams(needs_layout_passes=False),
        out_type=jax.ShapeDtypeStruct((n_rows // F_FIELDS, IN_DIM),
                                      jnp.float32),
        scratch_types=[
            pltpu.VMEM((rpw + 16,), jnp.int32),
            pltpu.VMEM((_SBUF * IN_DIM,), jnp.float32),
            pltpu.VMEM((_WIN, D, 128), jnp.float32),
            pltpu.SemaphoreType.DMA,
        ] + [pltpu.SemaphoreType.DMA] * _WIN,
    )


def _tc_body(c_ref, wk_ref, w1_ref, b1_ref, w2_ref, b2_ref, w3_ref, b3_ref,
             w4_ref, b4_ref, o_ref):
    c = c_ref[...]
    z = jnp.dot(c, wk_ref[...], preferred_element_type=jnp.float32)
    fw = 0.5 * jnp.sum(c * z, axis=1, keepdims=True)
    h = jnp.maximum(
        jnp.dot(c, w1_ref[...], preferred_element_type=jnp.float32)
        + b1_ref[...], 0.0)
    h = jnp.maximum(
        jnp.dot(h, w2_ref[...], preferred_element_type=jnp.float32)
        + b2_ref[...], 0.0)
    h = jnp.maximum(
        jnp.dot(h, w3_ref[...], preferred_element_type=jnp.float32)
        + b3_ref[...], 0.0)
    m = jnp.dot(h, w4_ref[...], preferred_element_type=jnp.float32) + b4_ref[...]
    o_ref[...] = jax.nn.sigmoid(fw + m)


_BB = 256


def _tc_call(c, wk, W1, b1, W2, b2, W3, b3, W4, b4):
    nb = c.shape[0]
    full = lambda i: (0, 0)
    return pl.pallas_call(
        _tc_body,
        grid=(nb // _BB,),
        in_specs=[
            pl.BlockSpec((_BB, IN_DIM), lambda i: (i, 0)),
            pl.BlockSpec((IN_DIM, IN_DIM), full),
            pl.BlockSpec((IN_DIM, 512), full),
            pl.BlockSpec((1, 512), full),
            pl.BlockSpec((512, 256), full),
            pl.BlockSpec((1, 256), full),
            pl.BlockSpec((256, 128), full),
            pl.BlockSpec((1, 128), full),
            pl.BlockSpec((128, 1), full),
            pl.BlockSpec((1, 1), full),
        ],
        out_specs=pl.BlockSpec((_BB, 1), lambda i: (i, 0)),
        out_shape=jax.ShapeDtypeStruct((nb, 1), jnp.float32),
    )(c, wk, W1, b1, W2, b2, W3, b3, W4, b4)


def kernel(x, tables, field_cov_W, W1, b1, W2, b2, W3, b3, W4, b4):
    idx_raw = x.astype(jnp.int32).reshape(-1)
    t3 = jnp.transpose(tables, (0, 2, 1))

    w_sym = (field_cov_W + field_cov_W.T) * 0.5
    w0 = w_sym * (1.0 - jnp.eye(F_FIELDS, dtype=jnp.float32))
    wk = (w0[:, None, :, None]
          * jnp.eye(D, dtype=jnp.float32)[None, :, None, :]
          ).reshape(IN_DIM, IN_DIM)

    c = _make_sc_gather(N_ROWS)(idx_raw, t3)
    out = _tc_call(c, wk, W1, b1.reshape(1, -1), W2, b2.reshape(1, -1),
                   W3, b3.reshape(1, -1), W4, b4.reshape(1, -1))
    return out.reshape(B)
